# baseline (speedup 1.0000x reference)
import functools

import jax
import jax.numpy as jnp
from jax.experimental import pallas as pl
from jax.experimental.pallas import tpu as pltpu
from jax.sharding import Mesh, PartitionSpec as P
import numpy as np

_VMEM_LIMIT = 56 * 1024 * 1024


def _gates(xg, hg, h, hh):
    r = jax.nn.sigmoid(xg[:, :hh] + hg[:, :hh])
    z = jax.nn.sigmoid(xg[:, hh:2 * hh] + hg[:, hh:2 * hh])
    n = jnp.tanh(xg[:, 2 * hh:] + r * hg[:, 2 * hh:])
    return (1.0 - z) * n + z * h


_C = 1


def _hilo_cat(w):
    w_hi = w.astype(jnp.bfloat16)
    w_lo = (w - w_hi.astype(jnp.float32)).astype(jnp.bfloat16)
    return jnp.concatenate([w_hi, w_lo], axis=0)


def _wide_hi(v):
    vh = v.astype(jnp.bfloat16)
    return jnp.concatenate([vh, vh], axis=1)


def _wide_lo(v):
    vh = v.astype(jnp.bfloat16)
    vl = (v - vh.astype(jnp.float32)).astype(jnp.bfloat16)
    return jnp.concatenate([vl, vl], axis=1)


def _dot2(v, w_ref):
    lhs = jnp.concatenate([_wide_hi(v), _wide_lo(v)], axis=0)
    d = jnp.dot(lhs, w_ref[...], preferred_element_type=jnp.float32)
    return d[0:1] + d[1:2]


def _rcopy(send_buf, recv_buf, slot, send_sem, recv_sem, partner):
    return pltpu.make_async_remote_copy(
        send_buf.at[slot], recv_buf.at[slot], send_sem, recv_sem,
        device_id=partner, device_id_type=pltpu.DeviceIdType.LOGICAL)


def _step(s, xg_t, h_ref, w_own, w_oth, bhh_ref, send_buf, recv_buf,
          send_sem, recv_sem, partner, hh):
    cw = hh // _C
    base = jnp.bitwise_and(s, 1) * _C
    base_prev = jnp.bitwise_and(s + 1, 1) * _C

    hg = _dot2(h_ref[...], w_own)

    for j in range(_C):
        @pl.when(s > 0)
        def _():
            _rcopy(send_buf, recv_buf, base_prev + j, send_sem, recv_sem,
                   partner).wait_recv()
        hg = hg + _dot2(recv_buf[base_prev + j], w_oth)

    hg = hg + bhh_ref[...]
    h_old = h_ref[...]
    for j in range(_C):
        r = jax.nn.sigmoid(xg_t[:, j * cw:(j + 1) * cw]
                           + hg[:, j * cw:(j + 1) * cw])
        z = jax.nn.sigmoid(xg_t[:, hh + j * cw: hh + (j + 1) * cw]
                           + hg[:, hh + j * cw: hh + (j + 1) * cw])
        n = jnp.tanh(xg_t[:, 2 * hh + j * cw: 2 * hh + (j + 1) * cw]
                     + r * hg[:, 2 * hh + j * cw: 2 * hh + (j + 1) * cw])
        h_j = (1.0 - z) * n + z * h_old[:, j * cw:(j + 1) * cw]
        h_ref[:, j * cw:(j + 1) * cw] = h_j

        @pl.when(s >= 2)
        def _():
            _rcopy(send_buf, recv_buf, base + j, send_sem, recv_sem,
                   partner).wait_send()
        send_buf[base + j] = h_j
        _rcopy(send_buf, recv_buf, base + j, send_sem, recv_sem,
               partner).start()
    return h_ref[...]


def _prologue(i, partner, h_ref, recv_buf, dma_pairs):
    @pl.when(i == 0)
    def _():
        copies = [pltpu.make_async_copy(src, dst, sem)
                  for src, dst, sem in dma_pairs]
        for cp in copies:
            cp.start()
        h_ref[...] = jnp.zeros_like(h_ref)
        recv_buf[...] = jnp.zeros_like(recv_buf)
        bar = pltpu.get_barrier_semaphore()
        pltpu.semaphore_signal(bar, device_id=partner,
                               device_id_type=pltpu.DeviceIdType.LOGICAL)
        pltpu.semaphore_wait(bar, 1)
        for cp in copies:
            cp.wait()


def _epilogue(i, ngrid, send_buf, recv_buf, send_sem, recv_sem, partner):
    @pl.when(i == ngrid - 1)
    def _():
        for k in range(2 * _C):
            _rcopy(send_buf, recv_buf, k, send_sem, recv_sem,
                   partner).wait_send()
        for k in range(_C):
            _rcopy(send_buf, recv_buf, k, send_sem, recv_sem,
                   partner).wait_recv()


def _e1_body(tb, ngrid, hh):
    def body(myid_ref, x_ref, wih_ref, bih_ref, bhh_ref, wown_hbm, woth_hbm,
             ys_ref, wown_v, woth_v, send_buf, recv_buf, h_ref,
             semw1, semw2, send_sem, recv_sem):
        i = pl.program_id(0)
        partner = 1 - myid_ref[0]
        _prologue(i, partner, h_ref, recv_buf,
                  [(wown_hbm, wown_v, semw1), (woth_hbm, woth_v, semw2)])

        def step(t, c):
            s = i * tb + t
            xg_t = x_ref[pl.ds(t, 1), :] * wih_ref[...] + bih_ref[...]
            h_new = _step(s, xg_t, h_ref, wown_v, woth_v, bhh_ref,
                          send_buf, recv_buf, send_sem, recv_sem, partner, hh)
            ys_ref[pl.ds(t, 1), :] = h_new
            return c

        jax.lax.fori_loop(0, tb, step, 0)
        _epilogue(i, ngrid, send_buf, recv_buf, send_sem, recv_sem, partner)
    return body


def _run_e1(myid, x, wih_row, bih, bhh, w_own, w_oth, *, tb):
    seq = x.shape[0]
    hh = w_own.shape[0] // 2
    g3 = w_own.shape[1]
    ngrid = seq // tb
    return pl.pallas_call(
        _e1_body(tb, ngrid, hh),
        grid=(ngrid,),
        in_specs=[
            pl.BlockSpec(memory_space=pltpu.SMEM),
            pl.BlockSpec((tb, 1), lambda i: (i, 0)),
            pl.BlockSpec((1, g3), lambda i: (0, 0)),
            pl.BlockSpec((1, g3), lambda i: (0, 0)),
            pl.BlockSpec((1, g3), lambda i: (0, 0)),
            pl.BlockSpec(memory_space=pl.ANY),
            pl.BlockSpec(memory_space=pl.ANY),
        ],
        out_specs=pl.BlockSpec((tb, hh), lambda i: (i, 0)),
        out_shape=jax.ShapeDtypeStruct((seq, hh), jnp.float32),
        scratch_shapes=[
            pltpu.VMEM((2 * hh, g3), jnp.bfloat16),
            pltpu.VMEM((2 * hh, g3), jnp.bfloat16),
            pltpu.VMEM((2 * _C, 1, hh // _C), jnp.float32),
            pltpu.VMEM((2 * _C, 1, hh // _C), jnp.float32),
            pltpu.VMEM((1, hh), jnp.float32),
            pltpu.SemaphoreType.DMA,
            pltpu.SemaphoreType.DMA,
            pltpu.SemaphoreType.DMA,
            pltpu.SemaphoreType.DMA,
        ],
        compiler_params=pltpu.CompilerParams(
            dimension_semantics=("arbitrary",),
            vmem_limit_bytes=_VMEM_LIMIT,
            collective_id=0,
        ),
        name="gru2_e1",
    )(myid, x, wih_row, bih, bhh, w_own, w_oth)


def _e2l_body(tb, emb):
    def body(ys1_ref, bih_ref, bhh_ref, wih_hbm, whh_hbm, ht_ref,
             wih_v, whh_v, xg_scr, h_ref, sem1, sem2):
        @pl.when(pl.program_id(0) == 0)
        def _():
            cp1 = pltpu.make_async_copy(wih_hbm, wih_v, sem1)
            cp2 = pltpu.make_async_copy(whh_hbm, whh_v, sem2)
            cp1.start()
            cp2.start()
            cp1.wait()
            cp2.wait()
            h_ref[...] = jnp.zeros_like(h_ref)

        ys1 = ys1_ref[...]
        xg_scr[...] = jnp.dot(_wide_hi(ys1), wih_v[...],
                              preferred_element_type=jnp.float32) + bih_ref[...]
        xg_scr[...] += jnp.dot(_wide_lo(ys1), wih_v[...],
                               preferred_element_type=jnp.float32)

        def step(t, c):
            hg = _dot2(h_ref[...], whh_v) + bhh_ref[...]
            h_ref[...] = _gates(xg_scr[pl.ds(t, 1), :], hg, h_ref[...], emb)
            return c

        jax.lax.fori_loop(0, tb, step, 0)
        ht_ref[...] = h_ref[...]
    return body


def _run_e2l(ys1, bih, bhh, wih_t, whh_t, *, tb):
    seq, hid = ys1.shape
    emb = whh_t.shape[0] // 2
    return pl.pallas_call(
        _e2l_body(tb, emb),
        grid=(seq // tb,),
        in_specs=[
            pl.BlockSpec((tb, hid), lambda i: (i, 0)),
            pl.BlockSpec((1, 3 * emb), lambda i: (0, 0)),
            pl.BlockSpec((1, 3 * emb), lambda i: (0, 0)),
            pl.BlockSpec(memory_space=pl.ANY),
            pl.BlockSpec(memory_space=pl.ANY),
        ],
        out_specs=pl.BlockSpec((1, emb), lambda i: (0, 0)),
        out_shape=jax.ShapeDtypeStruct((1, emb), jnp.float32),
        scratch_shapes=[
            pltpu.VMEM((2 * hid, 3 * emb), jnp.bfloat16),
            pltpu.VMEM((2 * emb, 3 * emb), jnp.bfloat16),
            pltpu.VMEM((tb, 3 * emb), jnp.float32),
            pltpu.VMEM((1, emb), jnp.float32),
            pltpu.SemaphoreType.DMA,
            pltpu.SemaphoreType.DMA,
        ],
        compiler_params=pltpu.CompilerParams(
            dimension_semantics=("arbitrary",),
            vmem_limit_bytes=_VMEM_LIMIT,
        ),
        name="gru_e2l",
    )(ys1, bih, bhh, wih_t, whh_t)


def _d1l_body(tb, emb):
    def body(emb_ref, bih_ref, bhh_ref, wih_hbm, whh_hbm, ys_ref,
             wih_v, whh_v, xg_ref, h_ref, sem1, sem2):
        @pl.when(pl.program_id(0) == 0)
        def _():
            cp1 = pltpu.make_async_copy(wih_hbm, wih_v, sem1)
            cp2 = pltpu.make_async_copy(whh_hbm, whh_v, sem2)
            cp1.start()
            cp2.start()
            cp1.wait()
            cp2.wait()
            h_ref[...] = jnp.zeros_like(h_ref)
            xg_ref[...] = _dot2(emb_ref[...], wih_v) + bih_ref[...]

        def step(t, c):
            hg = _dot2(h_ref[...], whh_v) + bhh_ref[...]
            h_new = _gates(xg_ref[...], hg, h_ref[...], emb)
            h_ref[...] = h_new
            ys_ref[pl.ds(t, 1), :] = h_new
            return c

        jax.lax.fori_loop(0, tb, step, 0)
    return body


def _run_d1l(emb_vec, bih, bhh, wih_t, whh_t, *, seq, tb):
    emb = whh_t.shape[0] // 2
    return pl.pallas_call(
        _d1l_body(tb, emb),
        grid=(seq // tb,),
        in_specs=[
            pl.BlockSpec((1, emb), lambda i: (0, 0)),
            pl.BlockSpec((1, 3 * emb), lambda i: (0, 0)),
            pl.BlockSpec((1, 3 * emb), lambda i: (0, 0)),
            pl.BlockSpec(memory_space=pl.ANY),
            pl.BlockSpec(memory_space=pl.ANY),
        ],
        out_specs=pl.BlockSpec((tb, emb), lambda i: (i, 0)),
        out_shape=jax.ShapeDtypeStruct((seq, emb), jnp.float32),
        scratch_shapes=[
            pltpu.VMEM((2 * emb, 3 * emb), jnp.bfloat16),
            pltpu.VMEM((2 * emb, 3 * emb), jnp.bfloat16),
            pltpu.VMEM((1, 3 * emb), jnp.float32),
            pltpu.VMEM((1, emb), jnp.float32),
            pltpu.SemaphoreType.DMA,
            pltpu.SemaphoreType.DMA,
        ],
        compiler_params=pltpu.CompilerParams(
            dimension_semantics=("arbitrary",),
            vmem_limit_bytes=_VMEM_LIMIT,
        ),
        name="gru_d1l",
    )(emb_vec, bih, bhh, wih_t, whh_t)


def _d2_body(tb, ngrid, hh):
    def body(myid_ref, ys2_ref, bih_ref, bhh_ref, outw_ref, outb_ref,
             wih_hbm, wown_hbm, woth_hbm,
             y_ref, wih_v, wown_v, woth_v, xg_scr, ys3_scr, send_buf,
             recv_buf, h_ref, semw1, semw2, semw3, send_sem, recv_sem):
        i = pl.program_id(0)
        partner = 1 - myid_ref[0]
        _prologue(i, partner, h_ref, recv_buf,
                  [(wih_hbm, wih_v, semw1), (wown_hbm, wown_v, semw2),
                   (woth_hbm, woth_v, semw3)])

        ys2 = ys2_ref[...]
        xg_scr[...] = jnp.dot(_wide_hi(ys2), wih_v[...],
                              preferred_element_type=jnp.float32) + bih_ref[...]
        xg_scr[...] += jnp.dot(_wide_lo(ys2), wih_v[...],
                               preferred_element_type=jnp.float32)

        def step(t, c):
            s = i * tb + t
            h_new = _step(s, xg_scr[pl.ds(t, 1), :], h_ref, wown_v, woth_v,
                          bhh_ref, send_buf, recv_buf, send_sem, recv_sem,
                          partner, hh)
            ys3_scr[pl.ds(t, 1), :] = h_new
            return c

        jax.lax.fori_loop(0, tb, step, 0)
        ys3 = ys3_scr[...]
        y_ref[...] = jnp.dot(_wide_hi(ys3), outw_ref[...],
                             preferred_element_type=jnp.float32) + outb_ref[...]
        y_ref[...] += jnp.dot(_wide_lo(ys3), outw_ref[...],
                              preferred_element_type=jnp.float32)
        _epilogue(i, ngrid, send_buf, recv_buf, send_sem, recv_sem, partner)
    return body


def _run_d2(myid, ys2, bih, bhh, out_wt, out_b_half, wih_t, w_own, w_oth, *, tb):
    seq, emb = ys2.shape
    hh = w_own.shape[0] // 2
    g3 = w_own.shape[1]
    ngrid = seq // tb
    return pl.pallas_call(
        _d2_body(tb, ngrid, hh),
        grid=(ngrid,),
        in_specs=[
            pl.BlockSpec(memory_space=pltpu.SMEM),
            pl.BlockSpec((tb, emb), lambda i: (i, 0)),
            pl.BlockSpec((1, g3), lambda i: (0, 0)),
            pl.BlockSpec((1, g3), lambda i: (0, 0)),
            pl.BlockSpec((2 * hh, 1), lambda i: (0, 0)),
            pl.BlockSpec((1, 1), lambda i: (0, 0)),
            pl.BlockSpec(memory_space=pl.ANY),
            pl.BlockSpec(memory_space=pl.ANY),
            pl.BlockSpec(memory_space=pl.ANY),
        ],
        out_specs=pl.BlockSpec((tb, 1), lambda i: (i, 0)),
        out_shape=jax.ShapeDtypeStruct((seq, 1), jnp.float32),
        scratch_shapes=[
            pltpu.VMEM((2 * emb, g3), jnp.bfloat16),
            pltpu.VMEM((2 * hh, g3), jnp.bfloat16),
            pltpu.VMEM((2 * hh, g3), jnp.bfloat16),
            pltpu.VMEM((tb, g3), jnp.float32),
            pltpu.VMEM((tb, hh), jnp.float32),
            pltpu.VMEM((2 * _C, 1, hh // _C), jnp.float32),
            pltpu.VMEM((2 * _C, 1, hh // _C), jnp.float32),
            pltpu.VMEM((1, hh), jnp.float32),
            pltpu.SemaphoreType.DMA,
            pltpu.SemaphoreType.DMA,
            pltpu.SemaphoreType.DMA,
            pltpu.SemaphoreType.DMA,
            pltpu.SemaphoreType.DMA,
        ],
        compiler_params=pltpu.CompilerParams(
            dimension_semantics=("arbitrary",),
            vmem_limit_bytes=_VMEM_LIMIT,
            collective_id=3,
        ),
        name="gru2_d2",
    )(myid, ys2, bih, bhh, out_wt, out_b_half, wih_t, w_own, w_oth)


def _pack_cols(w_t, h):
    hh = h // 2
    parts = []
    for c in range(2):
        parts.append(jnp.concatenate(
            [w_t[:, c * hh:(c + 1) * hh],
             w_t[:, h + c * hh: h + (c + 1) * hh],
             w_t[:, 2 * h + c * hh: 2 * h + (c + 1) * hh]], axis=1))
    return jnp.stack(parts)


def _pack_whh(whh, h):
    hh = h // 2
    packed = _pack_cols(whh.T, h)
    w_own = jnp.stack([_hilo_cat(packed[0, :hh]), _hilo_cat(packed[1, hh:])])
    w_oth = jnp.stack([_hilo_cat(packed[0, hh:]), _hilo_cat(packed[1, :hh])])
    return w_own, w_oth


def _pack_bias(b, h):
    return _pack_cols(b.reshape(1, -1), h)


def _two_core_fn(seq, hid, emb, tb, tb2):
    def fn(x, wih1, bih1, bhh1, whh1_own, whh1_oth,
           wih2, bih2, bhh2, whh2,
           wih3, bih3, bhh3, whh3,
           wih4, bih4, bhh4, whh4_own, whh4_oth,
           outw, outb_half):
        myid = jax.lax.axis_index("c").reshape((1,)).astype(jnp.int32)
        sq = lambda a: a[0]

        ys1_c = _run_e1(myid, x, sq(wih1), sq(bih1), sq(bhh1),
                        sq(whh1_own), sq(whh1_oth), tb=tb)
        ys1 = jax.lax.all_gather(ys1_c, "c", axis=1, tiled=True)

        emb_full = _run_e2l(ys1, bih2, bhh2, wih2, whh2, tb=tb2)
        ys2 = _run_d1l(emb_full, bih3, bhh3, wih3, whh3, seq=seq, tb=tb)

        y_part = _run_d2(myid, ys2, sq(bih4), sq(bhh4), sq(outw),
                         outb_half, sq(wih4), sq(whh4_own), sq(whh4_oth),
                         tb=tb2)
        return jax.lax.psum(y_part, "c")
    return fn


def kernel(x, e1_Wih, e1_Whh, e1_bih, e1_bhh,
           e2_Wih, e2_Whh, e2_bih, e2_bhh,
           d1_Wih, d1_Whh, d1_bih, d1_bhh,
           d2_Wih, d2_Whh, d2_bih, d2_bhh,
           out_W, out_b):
    bf16 = jnp.bfloat16
    seq = x.shape[0]
    hid = e1_Whh.shape[1]
    emb = e2_Whh.shape[1]
    hh, eh = hid // 2, emb // 2
    tb = min(512, seq)
    tb2 = min(256, seq)

    wih1 = _pack_cols(e1_Wih.T, hid)
    bih1, bhh1 = _pack_bias(e1_bih, hid), _pack_bias(e1_bhh, hid)
    whh1_own, whh1_oth = _pack_whh(e1_Whh, hid)

    wih2 = _hilo_cat(e2_Wih.T)
    bih2, bhh2 = e2_bih.reshape(1, -1), e2_bhh.reshape(1, -1)
    whh2 = _hilo_cat(e2_Whh.T)

    wih3 = _hilo_cat(d1_Wih.T)
    bih3, bhh3 = d1_bih.reshape(1, -1), d1_bhh.reshape(1, -1)
    whh3 = _hilo_cat(d1_Whh.T)

    wih4c = _pack_cols(d2_Wih.T, hid)
    wih4 = jnp.stack([_hilo_cat(wih4c[0]), _hilo_cat(wih4c[1])])
    bih4, bhh4 = _pack_bias(d2_bih, hid), _pack_bias(d2_bhh, hid)
    whh4_own, whh4_oth = _pack_whh(d2_Whh, hid)

    outw = jnp.stack([_hilo_cat(out_W.T[:hh]), _hilo_cat(out_W.T[hh:])])
    outb_half = (0.5 * out_b).reshape(1, 1).astype(jnp.float32)

    mesh = Mesh(np.array(jax.devices()[:2]), ("c",))
    shd = P("c")
    rep = P()
    fn = jax.shard_map(
        _two_core_fn(seq, hid, emb, tb, tb2),
        mesh=mesh,
        in_specs=(rep,
                  shd, shd, shd, shd, shd,
                  rep, rep, rep, rep,
                  rep, rep, rep, rep,
                  shd, shd, shd, shd, shd,
                  shd, rep),
        out_specs=rep,
        check_vma=False,
    )
    return fn(x.reshape(seq, 1),
              wih1, bih1, bhh1, whh1_own, whh1_oth,
              wih2, bih2, bhh2, whh2,
              wih3, bih3, bhh3, whh3,
              wih4, bih4, bhh4, whh4_own, whh4_oth,
              outw, outb_half)
